# baseline (speedup 1.0000x reference)
import functools
import math

import jax
import jax.numpy as jnp
import numpy as np
from jax import lax
from jax.experimental import pallas as pl
from jax.experimental.pallas import tpu as pltpu
from jax.experimental.pallas import tpu_sc as plsc

_K = 10
_N = 10000
_FIN = 128
_HID = 64
_NCLS = 32
_E = 320000

_NC, _NS = 2, 16
_NW = _NC * _NS
_CHUNK = 64
_NCH = 160
_NB = 2
_NOUT = _NCH // _NB
_HW = 128
_EPAD = _NCH * _NW * _CHUNK
_AR = 10112
_ZR = _AR // _NS
_ZCH = (128, 128, 128, 128, 120)
_BLK = 1000
_GRID = _N // _BLK


def _cheby_t(i, x):
    if i == 0:
        return 1.0
    t0, t1 = 1.0, x
    for _ in range(2, i + 1):
        t0, t1 = t1, 2.0 * x * t1 - t0
    return t1


def _interp_matrix(k):
    xs = [math.cos((k - j + 0.5) * math.pi / (k + 1)) for j in range(k + 1)]
    return np.array([[_cheby_t(i, xs[j]) for j in range(k + 1)]
                     for i in range(k + 1)], dtype=np.float32)

_M_INTERP = _interp_matrix(_K)

_MESH = plsc.VectorSubcoreMesh(core_axis_name="c", subcore_axis_name="s",
                               num_cores=_NC, num_subcores=_NS)


@functools.partial(
    pl.kernel,
    out_type=jax.ShapeDtypeStruct((_NC * _AR, _HW), jnp.float32),
    mesh=_MESH,
    scratch_types=[
        pltpu.VMEM((_NCH, _CHUNK), jnp.int32),
        pltpu.VMEM((_NCH, _CHUNK), jnp.int32),
        pltpu.VMEM((_CHUNK, _HW), jnp.float32),
        pltpu.VMEM_SHARED((_AR, _HW), jnp.float32),
    ],
)
def _prop_sc(zs, srcg, dsts, zrows, out, sidx, didx, rows, acc):
    c = lax.axis_index("c")
    s = lax.axis_index("s")
    w = c * _NS + s
    pltpu.sync_copy(srcg.at[w], sidx)
    pltpu.sync_copy(dsts.at[w], didx)
    off = 0
    for ln in _ZCH:
        pltpu.sync_copy(zrows.at[pl.ds(0, ln)],
                        acc.at[pl.ds(s * _ZR + off, ln)])
        off += ln
    plsc.subcore_barrier()

    def body(j, carry):
        pltpu.sync_copy(zs.at[sidx.at[j]], rows)
        pltpu.sync_copy(rows, acc.at[didx.at[j]], add=True)
        return carry

    lax.fori_loop(0, _NCH, body, 0)

    plsc.subcore_barrier()
    pltpu.sync_copy(acc.at[pl.ds(s * _ZR, _ZR)],
                    out.at[pl.ds(c * _AR + s * _ZR, _ZR)])


def _prologue_body(xr, w1r, b1r, degr, hr, zsr, dbr):
    h = jnp.maximum(
        jnp.dot(xr[...], w1r[...], preferred_element_type=jnp.float32)
        + b1r[...], 0.0)
    d = degr[0] + degr[1]
    dis = jnp.where(d > 0.0, lax.rsqrt(jnp.where(d > 0.0, d, 1.0)), 0.0)
    hr[...] = h
    zsr[:, :_HID] = dis[:, :_HID] * h
    zsr[:, _HID:] = jnp.zeros((_BLK, _HW - _HID), jnp.float32)
    dbr[...] = dis[:, :_HID]


_prologue_tc = pl.pallas_call(
    _prologue_body,
    grid=(_GRID,),
    in_specs=[
        pl.BlockSpec((_BLK, _FIN), lambda i: (i, 0)),
        pl.BlockSpec((_FIN, _HID), lambda i: (0, 0)),
        pl.BlockSpec((1, _HID), lambda i: (0, 0)),
        pl.BlockSpec((2, _BLK, _HW), lambda i: (0, i, 0)),
    ],
    out_specs=[
        pl.BlockSpec((_BLK, _HID), lambda i: (i, 0)),
        pl.BlockSpec((_BLK, _HW), lambda i: (i, 0)),
        pl.BlockSpec((_BLK, _HID), lambda i: (i, 0)),
    ],
    out_shape=[
        jax.ShapeDtypeStruct((_N, _HID), jnp.float32),
        jax.ShapeDtypeStruct((_N, _HW), jnp.float32),
        jax.ShapeDtypeStruct((_N, _HID), jnp.float32),
    ],
)


def _zs_store(zsr, db, t):
    zsr[:, :_HID] = db * t
    zsr[:, _HID:] = jnp.zeros((_BLK, _HW - _HID), jnp.float32)


def _step1_body(accr, dbr, h0r, scr, txr, outr, zsr):
    db = dbr[...]
    p = -db * (accr[0][:, :_HID] + accr[1][:, :_HID])
    txr[...] = p
    outr[...] = scr[0] * h0r[...] + scr[1] * p
    _zs_store(zsr, db, p)


_step1_tc = pl.pallas_call(
    _step1_body,
    grid=(_GRID,),
    in_specs=[
        pl.BlockSpec((2, _BLK, _HW), lambda i: (0, i, 0)),
        pl.BlockSpec((_BLK, _HID), lambda i: (i, 0)),
        pl.BlockSpec((_BLK, _HID), lambda i: (i, 0)),
        pl.BlockSpec(memory_space=pltpu.SMEM),
    ],
    out_specs=[
        pl.BlockSpec((_BLK, _HID), lambda i: (i, 0)),
        pl.BlockSpec((_BLK, _HID), lambda i: (i, 0)),
        pl.BlockSpec((_BLK, _HW), lambda i: (i, 0)),
    ],
    out_shape=[
        jax.ShapeDtypeStruct((_N, _HID), jnp.float32),
        jax.ShapeDtypeStruct((_N, _HID), jnp.float32),
        jax.ShapeDtypeStruct((_N, _HW), jnp.float32),
    ],
)


def _stepmid_body(accr, dbr, tx0r, outpr, cr, txr, outr, zsr):
    db = dbr[...]
    p = -db * (accr[0][:, :_HID] + accr[1][:, :_HID])
    t = 2.0 * p - tx0r[...]
    txr[...] = t
    outr[...] = outpr[...] + cr[0] * t
    _zs_store(zsr, db, t)


_stepmid_tc = pl.pallas_call(
    _stepmid_body,
    grid=(_GRID,),
    in_specs=[
        pl.BlockSpec((2, _BLK, _HW), lambda i: (0, i, 0)),
        pl.BlockSpec((_BLK, _HID), lambda i: (i, 0)),
        pl.BlockSpec((_BLK, _HID), lambda i: (i, 0)),
        pl.BlockSpec((_BLK, _HID), lambda i: (i, 0)),
        pl.BlockSpec(memory_space=pltpu.SMEM),
    ],
    out_specs=[
        pl.BlockSpec((_BLK, _HID), lambda i: (i, 0)),
        pl.BlockSpec((_BLK, _HID), lambda i: (i, 0)),
        pl.BlockSpec((_BLK, _HW), lambda i: (i, 0)),
    ],
    out_shape=[
        jax.ShapeDtypeStruct((_N, _HID), jnp.float32),
        jax.ShapeDtypeStruct((_N, _HID), jnp.float32),
        jax.ShapeDtypeStruct((_N, _HW), jnp.float32),
    ],
)


def _steplast_body(accr, dbr, tx0r, outpr, cr, w2r, b2r, houtr, yr):
    p = -dbr[...] * (accr[0][:, :_HID] + accr[1][:, :_HID])
    t = 2.0 * p - tx0r[...]
    h = outpr[...] + cr[0] * t
    houtr[...] = h
    yr[...] = jnp.dot(h, w2r[...], preferred_element_type=jnp.float32) + b2r[...]


_steplast_tc = pl.pallas_call(
    _steplast_body,
    grid=(_GRID,),
    in_specs=[
        pl.BlockSpec((2, _BLK, _HW), lambda i: (0, i, 0)),
        pl.BlockSpec((_BLK, _HID), lambda i: (i, 0)),
        pl.BlockSpec((_BLK, _HID), lambda i: (i, 0)),
        pl.BlockSpec((_BLK, _HID), lambda i: (i, 0)),
        pl.BlockSpec(memory_space=pltpu.SMEM),
        pl.BlockSpec((_HID, _NCLS), lambda i: (0, 0)),
        pl.BlockSpec((1, _NCLS), lambda i: (0, 0)),
    ],
    out_specs=[
        pl.BlockSpec((_BLK, _HID), lambda i: (i, 0)),
        pl.BlockSpec((_BLK, _NCLS), lambda i: (i, 0)),
    ],
    out_shape=[
        jax.ShapeDtypeStruct((_N, _HID), jnp.float32),
        jax.ShapeDtypeStruct((_N, _NCLS), jnp.float32),
    ],
)


def kernel(edge_index, x, W1, b1, W2, b2, temp):
    src = edge_index[0]
    dst = edge_index[1]
    npad = _EPAD - _E
    srcg = jnp.concatenate([src, jnp.zeros((npad,), jnp.int32)]).reshape(
        _NW, _NCH, _CHUNK)
    trash = jnp.full((npad,), _N, jnp.int32)
    dsts = jnp.concatenate([dst, trash]).reshape(_NW, _NCH, _CHUNK)
    srcs = jnp.concatenate([src, trash]).reshape(_NW, _NCH, _CHUNK)

    onesm = jnp.ones((_N, _HW), jnp.float32)
    zrows = jnp.zeros((128, _HW), jnp.float32)

    coe = (2.0 / (_K + 1)) * (jnp.asarray(_M_INTERP) @ jnp.maximum(temp, 0.0))

    degp = _prop_sc(onesm, srcg, srcs, zrows).reshape(_NC, _AR, _HW)
    h0, zs, db = _prologue_tc(x, W1, b1.reshape(1, _HID), degp)

    acc = _prop_sc(zs, srcg, dsts, zrows).reshape(_NC, _AR, _HW)
    sc1 = jnp.stack([coe[0] * 0.5, coe[1]])
    tx1, out, zs = _step1_tc(acc, db, h0, sc1)
    tx0 = h0
    for i in range(2, _K):
        acc = _prop_sc(zs, srcg, dsts, zrows).reshape(_NC, _AR, _HW)
        tx2, out, zs = _stepmid_tc(acc, db, tx0, out, coe[i:i + 1])
        tx0 = tx1
        tx1 = tx2
    acc = _prop_sc(zs, srcg, dsts, zrows).reshape(_NC, _AR, _HW)
    h_out, y = _steplast_tc(acc, db, tx0, out, coe[_K:_K + 1], W2,
                            b2.reshape(1, _NCLS))
    return (y, h_out)


# speedup vs baseline: 1.1737x; 1.1737x over previous
import functools
import math

import jax
import jax.numpy as jnp
import numpy as np
from jax import lax
from jax.experimental import pallas as pl
from jax.experimental.pallas import tpu as pltpu
from jax.experimental.pallas import tpu_sc as plsc

_K = 10
_N = 10000
_FIN = 128
_HID = 64
_NCLS = 32
_E = 320000

_NC, _NS = 2, 16
_NW = _NC * _NS
_CHUNK = 64
_NCH = 160
_NB = 4
_IBC = 16
_NBLK = _NCH // _IBC
_IOUT = _IBC // _NB
_HW = 128
_EPAD = _NCH * _NW * _CHUNK
_AR = 10112
_ZR = _AR // _NS
_ZCH = (128, 128, 128, 128, 120)
_BLK = 1000
_GRID = _N // _BLK


def _cheby_t(i, x):
    if i == 0:
        return 1.0
    t0, t1 = 1.0, x
    for _ in range(2, i + 1):
        t0, t1 = t1, 2.0 * x * t1 - t0
    return t1


def _interp_matrix(k):
    xs = [math.cos((k - j + 0.5) * math.pi / (k + 1)) for j in range(k + 1)]
    return np.array([[_cheby_t(i, xs[j]) for j in range(k + 1)]
                     for i in range(k + 1)], dtype=np.float32)

_M_INTERP = _interp_matrix(_K)

_MESH = plsc.VectorSubcoreMesh(core_axis_name="c", subcore_axis_name="s",
                               num_cores=_NC, num_subcores=_NS)


@functools.partial(
    pl.kernel,
    out_type=jax.ShapeDtypeStruct((_NC * _AR, _HW), jnp.float32),
    mesh=_MESH,
    scratch_types=[
        pltpu.VMEM((2, _IBC, _CHUNK), jnp.int32),
        pltpu.VMEM((2, _IBC, _CHUNK), jnp.int32),
        pltpu.VMEM((_NB * _CHUNK, _HW), jnp.float32),
        pltpu.VMEM_SHARED((_AR, _HW), jnp.float32),
        pltpu.SemaphoreType.DMA,
        pltpu.SemaphoreType.DMA,
        pltpu.SemaphoreType.DMA,
        pltpu.SemaphoreType.DMA,
        pltpu.SemaphoreType.DMA,
    ],
)
def _prop_sc(zs, srcg, dsts, zrows, out, sidx, didx, rows, acc,
             gsem0, gsem1, gsem2, gsem3, isem):
    c = lax.axis_index("c")
    s = lax.axis_index("s")
    w = c * _NS + s

    def _iblk(k):
        return pl.ds(k * _IBC, _IBC)

    pltpu.async_copy(srcg.at[w, _iblk(0)], sidx.at[0], isem)
    pltpu.async_copy(dsts.at[w, _iblk(0)], didx.at[0], isem)

    off = 0
    for ln in _ZCH:
        pltpu.sync_copy(zrows.at[pl.ds(0, ln)],
                        acc.at[pl.ds(s * _ZR + off, ln)])
        off += ln
    plsc.subcore_barrier()

    gsems = (gsem0, gsem1, gsem2, gsem3)

    def _buf(b):
        return rows.at[pl.ds(b * _CHUNK, _CHUNK)]

    def blk_body(k, carry):
        p = k & 1
        pltpu.make_async_copy(srcg.at[w, _iblk(k)], sidx.at[p], isem).wait()
        pltpu.make_async_copy(dsts.at[w, _iblk(k)], didx.at[p], isem).wait()
        kn = jnp.minimum(k + 1, _NBLK - 1)
        q = kn & 1
        pltpu.async_copy(srcg.at[w, _iblk(kn)], sidx.at[q], isem)
        pltpu.async_copy(dsts.at[w, _iblk(kn)], didx.at[q], isem)

        for b in range(_NB):
            pltpu.async_copy(zs.at[sidx.at[p, b]], _buf(b), gsems[b])

        def body(g, carry2):
            j0 = g * _NB
            for b in range(_NB):
                j = j0 + b
                pltpu.make_async_copy(zs.at[sidx.at[p, j]], _buf(b),
                                      gsems[b]).wait()
                pltpu.sync_copy(_buf(b), acc.at[didx.at[p, j]], add=True)
                pltpu.async_copy(zs.at[sidx.at[p, j + _NB]], _buf(b), gsems[b])
            return carry2

        lax.fori_loop(0, _IOUT - 1, body, 0)
        for b in range(_NB):
            j = (_IOUT - 1) * _NB + b
            pltpu.make_async_copy(zs.at[sidx.at[p, j]], _buf(b),
                                  gsems[b]).wait()
            pltpu.sync_copy(_buf(b), acc.at[didx.at[p, j]], add=True)
        return carry

    lax.fori_loop(0, _NBLK, blk_body, 0)
    pltpu.make_async_copy(srcg.at[w, _iblk(_NBLK - 1)],
                          sidx.at[(_NBLK - 1) & 1], isem).wait()
    pltpu.make_async_copy(dsts.at[w, _iblk(_NBLK - 1)],
                          didx.at[(_NBLK - 1) & 1], isem).wait()

    plsc.subcore_barrier()
    pltpu.sync_copy(acc.at[pl.ds(s * _ZR, _ZR)],
                    out.at[pl.ds(c * _AR + s * _ZR, _ZR)])


def _prologue_body(xr, w1r, b1r, degr, hr, zsr, dbr):
    h = jnp.maximum(
        jnp.dot(xr[...], w1r[...], preferred_element_type=jnp.float32)
        + b1r[...], 0.0)
    d = degr[0] + degr[1]
    dis = jnp.where(d > 0.0, lax.rsqrt(jnp.where(d > 0.0, d, 1.0)), 0.0)
    hr[...] = h
    zsr[:, :_HID] = dis[:, :_HID] * h
    zsr[:, _HID:] = jnp.zeros((_BLK, _HW - _HID), jnp.float32)
    dbr[...] = dis[:, :_HID]


_prologue_tc = pl.pallas_call(
    _prologue_body,
    grid=(_GRID,),
    in_specs=[
        pl.BlockSpec((_BLK, _FIN), lambda i: (i, 0)),
        pl.BlockSpec((_FIN, _HID), lambda i: (0, 0)),
        pl.BlockSpec((1, _HID), lambda i: (0, 0)),
        pl.BlockSpec((2, _BLK, _HW), lambda i: (0, i, 0)),
    ],
    out_specs=[
        pl.BlockSpec((_BLK, _HID), lambda i: (i, 0)),
        pl.BlockSpec((_BLK, _HW), lambda i: (i, 0)),
        pl.BlockSpec((_BLK, _HID), lambda i: (i, 0)),
    ],
    out_shape=[
        jax.ShapeDtypeStruct((_N, _HID), jnp.float32),
        jax.ShapeDtypeStruct((_N, _HW), jnp.float32),
        jax.ShapeDtypeStruct((_N, _HID), jnp.float32),
    ],
)


def _zs_store(zsr, db, t):
    zsr[:, :_HID] = db * t
    zsr[:, _HID:] = jnp.zeros((_BLK, _HW - _HID), jnp.float32)


def _step1_body(accr, dbr, h0r, scr, txr, outr, zsr):
    db = dbr[...]
    p = -db * (accr[0][:, :_HID] + accr[1][:, :_HID])
    txr[...] = p
    outr[...] = scr[0] * h0r[...] + scr[1] * p
    _zs_store(zsr, db, p)


_step1_tc = pl.pallas_call(
    _step1_body,
    grid=(_GRID,),
    in_specs=[
        pl.BlockSpec((2, _BLK, _HW), lambda i: (0, i, 0)),
        pl.BlockSpec((_BLK, _HID), lambda i: (i, 0)),
        pl.BlockSpec((_BLK, _HID), lambda i: (i, 0)),
        pl.BlockSpec(memory_space=pltpu.SMEM),
    ],
    out_specs=[
        pl.BlockSpec((_BLK, _HID), lambda i: (i, 0)),
        pl.BlockSpec((_BLK, _HID), lambda i: (i, 0)),
        pl.BlockSpec((_BLK, _HW), lambda i: (i, 0)),
    ],
    out_shape=[
        jax.ShapeDtypeStruct((_N, _HID), jnp.float32),
        jax.ShapeDtypeStruct((_N, _HID), jnp.float32),
        jax.ShapeDtypeStruct((_N, _HW), jnp.float32),
    ],
)


def _stepmid_body(accr, dbr, tx0r, outpr, cr, txr, outr, zsr):
    db = dbr[...]
    p = -db * (accr[0][:, :_HID] + accr[1][:, :_HID])
    t = 2.0 * p - tx0r[...]
    txr[...] = t
    outr[...] = outpr[...] + cr[0] * t
    _zs_store(zsr, db, t)


_stepmid_tc = pl.pallas_call(
    _stepmid_body,
    grid=(_GRID,),
    in_specs=[
        pl.BlockSpec((2, _BLK, _HW), lambda i: (0, i, 0)),
        pl.BlockSpec((_BLK, _HID), lambda i: (i, 0)),
        pl.BlockSpec((_BLK, _HID), lambda i: (i, 0)),
        pl.BlockSpec((_BLK, _HID), lambda i: (i, 0)),
        pl.BlockSpec(memory_space=pltpu.SMEM),
    ],
    out_specs=[
        pl.BlockSpec((_BLK, _HID), lambda i: (i, 0)),
        pl.BlockSpec((_BLK, _HID), lambda i: (i, 0)),
        pl.BlockSpec((_BLK, _HW), lambda i: (i, 0)),
    ],
    out_shape=[
        jax.ShapeDtypeStruct((_N, _HID), jnp.float32),
        jax.ShapeDtypeStruct((_N, _HID), jnp.float32),
        jax.ShapeDtypeStruct((_N, _HW), jnp.float32),
    ],
)


def _steplast_body(accr, dbr, tx0r, outpr, cr, w2r, b2r, houtr, yr):
    p = -dbr[...] * (accr[0][:, :_HID] + accr[1][:, :_HID])
    t = 2.0 * p - tx0r[...]
    h = outpr[...] + cr[0] * t
    houtr[...] = h
    yr[...] = jnp.dot(h, w2r[...], preferred_element_type=jnp.float32) + b2r[...]


_steplast_tc = pl.pallas_call(
    _steplast_body,
    grid=(_GRID,),
    in_specs=[
        pl.BlockSpec((2, _BLK, _HW), lambda i: (0, i, 0)),
        pl.BlockSpec((_BLK, _HID), lambda i: (i, 0)),
        pl.BlockSpec((_BLK, _HID), lambda i: (i, 0)),
        pl.BlockSpec((_BLK, _HID), lambda i: (i, 0)),
        pl.BlockSpec(memory_space=pltpu.SMEM),
        pl.BlockSpec((_HID, _NCLS), lambda i: (0, 0)),
        pl.BlockSpec((1, _NCLS), lambda i: (0, 0)),
    ],
    out_specs=[
        pl.BlockSpec((_BLK, _HID), lambda i: (i, 0)),
        pl.BlockSpec((_BLK, _NCLS), lambda i: (i, 0)),
    ],
    out_shape=[
        jax.ShapeDtypeStruct((_N, _HID), jnp.float32),
        jax.ShapeDtypeStruct((_N, _NCLS), jnp.float32),
    ],
)


def kernel(edge_index, x, W1, b1, W2, b2, temp):
    src = edge_index[0]
    dst = edge_index[1]
    npad = _EPAD - _E
    srcg = jnp.concatenate([src, jnp.zeros((npad,), jnp.int32)]).reshape(
        _NW, _NCH, _CHUNK)
    trash = jnp.full((npad,), _N, jnp.int32)
    dsts = jnp.concatenate([dst, trash]).reshape(_NW, _NCH, _CHUNK)
    srcs = jnp.concatenate([src, trash]).reshape(_NW, _NCH, _CHUNK)

    onesm = jnp.ones((_N, _HW), jnp.float32)
    zrows = jnp.zeros((128, _HW), jnp.float32)

    coe = (2.0 / (_K + 1)) * (jnp.asarray(_M_INTERP) @ jnp.maximum(temp, 0.0))

    degp = _prop_sc(onesm, srcg, srcs, zrows).reshape(_NC, _AR, _HW)
    h0, zs, db = _prologue_tc(x, W1, b1.reshape(1, _HID), degp)

    acc = _prop_sc(zs, srcg, dsts, zrows).reshape(_NC, _AR, _HW)
    sc1 = jnp.stack([coe[0] * 0.5, coe[1]])
    tx1, out, zs = _step1_tc(acc, db, h0, sc1)
    tx0 = h0
    for i in range(2, _K):
        acc = _prop_sc(zs, srcg, dsts, zrows).reshape(_NC, _AR, _HW)
        tx2, out, zs = _stepmid_tc(acc, db, tx0, out, coe[i:i + 1])
        tx0 = tx1
        tx1 = tx2
    acc = _prop_sc(zs, srcg, dsts, zrows).reshape(_NC, _AR, _HW)
    h_out, y = _steplast_tc(acc, db, tx0, out, coe[_K:_K + 1], W2,
                            b2.reshape(1, _NCLS))
    return (y, h_out)


# speedup vs baseline: 1.2180x; 1.0378x over previous
import functools
import math

import jax
import jax.numpy as jnp
import numpy as np
from jax import lax
from jax.experimental import pallas as pl
from jax.experimental.pallas import tpu as pltpu
from jax.experimental.pallas import tpu_sc as plsc

_K = 10
_N = 10000
_FIN = 128
_HID = 64
_NCLS = 32
_E = 320000

_NC, _NS = 2, 16
_NW = _NC * _NS
_CHUNK = 128
_NCH = 80
_NB = 2
_IBC = 16
_NIB = _NCH // _IBC
_IOUT = _IBC // _NB
_HW = 128
_EPAD = _NCH * _NW * _CHUNK
_AR = 10112
_ZR = _AR // _NS
_ZCH = (128, 128, 128, 128, 120)
_BLK = 1000
_GRID = _N // _BLK


def _cheby_t(i, x):
    if i == 0:
        return 1.0
    t0, t1 = 1.0, x
    for _ in range(2, i + 1):
        t0, t1 = t1, 2.0 * x * t1 - t0
    return t1


def _interp_matrix(k):
    xs = [math.cos((k - j + 0.5) * math.pi / (k + 1)) for j in range(k + 1)]
    return np.array([[_cheby_t(i, xs[j]) for j in range(k + 1)]
                     for i in range(k + 1)], dtype=np.float32)

_M_INTERP = _interp_matrix(_K)

_MESH = plsc.VectorSubcoreMesh(core_axis_name="c", subcore_axis_name="s",
                               num_cores=_NC, num_subcores=_NS)


@functools.partial(
    pl.kernel,
    out_type=jax.ShapeDtypeStruct((_NC * _AR, _HW), jnp.float32),
    mesh=_MESH,
    scratch_types=[
        pltpu.VMEM((2, _IBC, _CHUNK), jnp.int32),
        pltpu.VMEM((2, _IBC, _CHUNK), jnp.int32),
        pltpu.VMEM((_NB * _CHUNK, _HW), jnp.float32),
        pltpu.VMEM_SHARED((_AR, _HW), jnp.float32),
        pltpu.SemaphoreType.DMA,
        pltpu.SemaphoreType.DMA,
        pltpu.SemaphoreType.DMA,
    ],
)
def _prop_sc(zs, srcg, dsts, zrows, out, sidx, didx, rows, acc,
             gsem0, gsem1, isem):
    c = lax.axis_index("c")
    s = lax.axis_index("s")
    w = c * _NS + s

    def _iblk(k):
        return pl.ds(k * _IBC, _IBC)

    pltpu.async_copy(srcg.at[w, _iblk(0)], sidx.at[0], isem)
    pltpu.async_copy(dsts.at[w, _iblk(0)], didx.at[0], isem)

    off = 0
    for ln in _ZCH:
        pltpu.sync_copy(zrows.at[pl.ds(0, ln)],
                        acc.at[pl.ds(s * _ZR + off, ln)])
        off += ln
    plsc.subcore_barrier()

    gsems = (gsem0, gsem1)

    def _buf(b):
        return rows.at[pl.ds(b * _CHUNK, _CHUNK)]

    for k in range(_NIB):
        p = k & 1
        pltpu.make_async_copy(srcg.at[w, _iblk(k)], sidx.at[p], isem).wait()
        pltpu.make_async_copy(dsts.at[w, _iblk(k)], didx.at[p], isem).wait()
        if k + 1 < _NIB:
            q = (k + 1) & 1
            pltpu.async_copy(srcg.at[w, _iblk(k + 1)], sidx.at[q], isem)
            pltpu.async_copy(dsts.at[w, _iblk(k + 1)], didx.at[q], isem)

        for b in range(_NB):
            pltpu.async_copy(zs.at[sidx.at[p, b]], _buf(b), gsems[b])

        def body(g, carry, p=p):
            j0 = g * _NB
            for b in range(_NB):
                j = j0 + b
                pltpu.make_async_copy(zs.at[sidx.at[p, j]], _buf(b),
                                      gsems[b]).wait()
                pltpu.sync_copy(_buf(b), acc.at[didx.at[p, j]], add=True)
                pltpu.async_copy(zs.at[sidx.at[p, j + _NB]], _buf(b), gsems[b])
            return carry

        lax.fori_loop(0, _IOUT - 1, body, 0)
        for b in range(_NB):
            j = (_IOUT - 1) * _NB + b
            pltpu.make_async_copy(zs.at[sidx.at[p, j]], _buf(b),
                                  gsems[b]).wait()
            pltpu.sync_copy(_buf(b), acc.at[didx.at[p, j]], add=True)

    plsc.subcore_barrier()
    pltpu.sync_copy(acc.at[pl.ds(s * _ZR, _ZR)],
                    out.at[pl.ds(c * _AR + s * _ZR, _ZR)])


def _prologue_body(xr, w1r, b1r, degr, hr, zsr, dbr):
    h = jnp.maximum(
        jnp.dot(xr[...], w1r[...], preferred_element_type=jnp.float32)
        + b1r[...], 0.0)
    d = degr[0] + degr[1]
    dis = jnp.where(d > 0.0, lax.rsqrt(jnp.where(d > 0.0, d, 1.0)), 0.0)
    hr[...] = h
    zsr[:, :_HID] = dis[:, :_HID] * h
    zsr[:, _HID:] = jnp.zeros((_BLK, _HW - _HID), jnp.float32)
    dbr[...] = dis[:, :_HID]


_prologue_tc = pl.pallas_call(
    _prologue_body,
    grid=(_GRID,),
    in_specs=[
        pl.BlockSpec((_BLK, _FIN), lambda i: (i, 0)),
        pl.BlockSpec((_FIN, _HID), lambda i: (0, 0)),
        pl.BlockSpec((1, _HID), lambda i: (0, 0)),
        pl.BlockSpec((2, _BLK, _HW), lambda i: (0, i, 0)),
    ],
    out_specs=[
        pl.BlockSpec((_BLK, _HID), lambda i: (i, 0)),
        pl.BlockSpec((_BLK, _HW), lambda i: (i, 0)),
        pl.BlockSpec((_BLK, _HID), lambda i: (i, 0)),
    ],
    out_shape=[
        jax.ShapeDtypeStruct((_N, _HID), jnp.float32),
        jax.ShapeDtypeStruct((_N, _HW), jnp.float32),
        jax.ShapeDtypeStruct((_N, _HID), jnp.float32),
    ],
)


def _zs_store(zsr, db, t):
    zsr[:, :_HID] = db * t
    zsr[:, _HID:] = jnp.zeros((_BLK, _HW - _HID), jnp.float32)


def _step1_body(accr, dbr, h0r, scr, txr, outr, zsr):
    db = dbr[...]
    p = -db * (accr[0][:, :_HID] + accr[1][:, :_HID])
    txr[...] = p
    outr[...] = scr[0] * h0r[...] + scr[1] * p
    _zs_store(zsr, db, p)


_step1_tc = pl.pallas_call(
    _step1_body,
    grid=(_GRID,),
    in_specs=[
        pl.BlockSpec((2, _BLK, _HW), lambda i: (0, i, 0)),
        pl.BlockSpec((_BLK, _HID), lambda i: (i, 0)),
        pl.BlockSpec((_BLK, _HID), lambda i: (i, 0)),
        pl.BlockSpec(memory_space=pltpu.SMEM),
    ],
    out_specs=[
        pl.BlockSpec((_BLK, _HID), lambda i: (i, 0)),
        pl.BlockSpec((_BLK, _HID), lambda i: (i, 0)),
        pl.BlockSpec((_BLK, _HW), lambda i: (i, 0)),
    ],
    out_shape=[
        jax.ShapeDtypeStruct((_N, _HID), jnp.float32),
        jax.ShapeDtypeStruct((_N, _HID), jnp.float32),
        jax.ShapeDtypeStruct((_N, _HW), jnp.float32),
    ],
)


def _stepmid_body(accr, dbr, tx0r, outpr, cr, txr, outr, zsr):
    db = dbr[...]
    p = -db * (accr[0][:, :_HID] + accr[1][:, :_HID])
    t = 2.0 * p - tx0r[...]
    txr[...] = t
    outr[...] = outpr[...] + cr[0] * t
    _zs_store(zsr, db, t)


_stepmid_tc = pl.pallas_call(
    _stepmid_body,
    grid=(_GRID,),
    in_specs=[
        pl.BlockSpec((2, _BLK, _HW), lambda i: (0, i, 0)),
        pl.BlockSpec((_BLK, _HID), lambda i: (i, 0)),
        pl.BlockSpec((_BLK, _HID), lambda i: (i, 0)),
        pl.BlockSpec((_BLK, _HID), lambda i: (i, 0)),
        pl.BlockSpec(memory_space=pltpu.SMEM),
    ],
    out_specs=[
        pl.BlockSpec((_BLK, _HID), lambda i: (i, 0)),
        pl.BlockSpec((_BLK, _HID), lambda i: (i, 0)),
        pl.BlockSpec((_BLK, _HW), lambda i: (i, 0)),
    ],
    out_shape=[
        jax.ShapeDtypeStruct((_N, _HID), jnp.float32),
        jax.ShapeDtypeStruct((_N, _HID), jnp.float32),
        jax.ShapeDtypeStruct((_N, _HW), jnp.float32),
    ],
)


def _steplast_body(accr, dbr, tx0r, outpr, cr, w2r, b2r, houtr, yr):
    p = -dbr[...] * (accr[0][:, :_HID] + accr[1][:, :_HID])
    t = 2.0 * p - tx0r[...]
    h = outpr[...] + cr[0] * t
    houtr[...] = h
    yr[...] = jnp.dot(h, w2r[...], preferred_element_type=jnp.float32) + b2r[...]


_steplast_tc = pl.pallas_call(
    _steplast_body,
    grid=(_GRID,),
    in_specs=[
        pl.BlockSpec((2, _BLK, _HW), lambda i: (0, i, 0)),
        pl.BlockSpec((_BLK, _HID), lambda i: (i, 0)),
        pl.BlockSpec((_BLK, _HID), lambda i: (i, 0)),
        pl.BlockSpec((_BLK, _HID), lambda i: (i, 0)),
        pl.BlockSpec(memory_space=pltpu.SMEM),
        pl.BlockSpec((_HID, _NCLS), lambda i: (0, 0)),
        pl.BlockSpec((1, _NCLS), lambda i: (0, 0)),
    ],
    out_specs=[
        pl.BlockSpec((_BLK, _HID), lambda i: (i, 0)),
        pl.BlockSpec((_BLK, _NCLS), lambda i: (i, 0)),
    ],
    out_shape=[
        jax.ShapeDtypeStruct((_N, _HID), jnp.float32),
        jax.ShapeDtypeStruct((_N, _NCLS), jnp.float32),
    ],
)


def kernel(edge_index, x, W1, b1, W2, b2, temp):
    src = edge_index[0]
    dst = edge_index[1]
    npad = _EPAD - _E
    srcg = jnp.concatenate([src, jnp.zeros((npad,), jnp.int32)]).reshape(
        _NW, _NCH, _CHUNK)
    trash = jnp.full((npad,), _N, jnp.int32)
    dsts = jnp.concatenate([dst, trash]).reshape(_NW, _NCH, _CHUNK)
    srcs = jnp.concatenate([src, trash]).reshape(_NW, _NCH, _CHUNK)

    onesm = jnp.ones((_N, _HW), jnp.float32)
    zrows = jnp.zeros((128, _HW), jnp.float32)

    coe = (2.0 / (_K + 1)) * (jnp.asarray(_M_INTERP) @ jnp.maximum(temp, 0.0))

    degp = _prop_sc(onesm, srcg, srcs, zrows).reshape(_NC, _AR, _HW)
    h0, zs, db = _prologue_tc(x, W1, b1.reshape(1, _HID), degp)

    acc = _prop_sc(zs, srcg, dsts, zrows).reshape(_NC, _AR, _HW)
    sc1 = jnp.stack([coe[0] * 0.5, coe[1]])
    tx1, out, zs = _step1_tc(acc, db, h0, sc1)
    tx0 = h0
    for i in range(2, _K):
        acc = _prop_sc(zs, srcg, dsts, zrows).reshape(_NC, _AR, _HW)
        tx2, out, zs = _stepmid_tc(acc, db, tx0, out, coe[i:i + 1])
        tx0 = tx1
        tx1 = tx2
    acc = _prop_sc(zs, srcg, dsts, zrows).reshape(_NC, _AR, _HW)
    h_out, y = _steplast_tc(acc, db, tx0, out, coe[_K:_K + 1], W2,
                            b2.reshape(1, _NCLS))
    return (y, h_out)
